# baseline (speedup 1.0000x reference)
import jax
import jax.numpy as jnp
from jax import lax
from jax.experimental import pallas as pl
from jax.experimental.pallas import tpu as pltpu
from jax.experimental.pallas import tpu_sc as plsc

N_USER = 5000
N_ITEM = 5000
N = N_USER + N_ITEM
D = 128
DH = 64
E = 320000
NC = 2
NS = 16
L = 16

K = 80
NBUF = 6
LA = 4
CHT = 252
SCH = 42
NSC = CHT // SCH
GRP = SCH // NBUF
E_PAD = NS * CHT * K
CR = 80
NB = N // CR
NQ = DH // L


def _body(tbl, srcm, dstm, wm, out,
          src_v, dst_v, w_v, gb0, gb1, gb2, gb3, gb4, gb5, cb0, cb1,
          acc1, acc2,
          gs0, gs1, gs2, gs3, gs4, gs5, ss0, ss1, ss2, ss3, ss4, ss5):
    cid = lax.axis_index("c")
    sid = lax.axis_index("s")
    gbufs = (gb0, gb1, gb2, gb3, gb4, gb5)
    gsems = (gs0, gs1, gs2, gs3, gs4, gs5)
    ssems = (ss0, ss1, ss2, ss3, ss4, ss5)

    zero16 = jnp.zeros((L,), jnp.float32)

    def zrow(r, carry):
        for q in range(NQ):
            cb0[r, pl.ds(q * L, L)] = zero16
        return carry

    lax.fori_loop(0, CR, zrow, 0)

    def for_blocks(fn):
        for t in range((NB + NS - 1) // NS):
            bb = t * NS + sid

            @pl.when(bb < NB)
            def _():
                fn(pl.ds(bb * CR, CR))

    def stage0(rs):
        pltpu.sync_copy(tbl.at[cid].at[rs], acc2.at[rs])
        pltpu.sync_copy(cb0, acc1.at[rs])

    for_blocks(stage0)
    plsc.subcore_barrier()

    def scale_chunk(j, b):
        buf = gbufs[b]
        for g in range(K // L):
            wvec = w_v[j, pl.ds(g * L, L)]
            for l in range(L):
                w = wvec[l]
                e = g * L + l
                for q in range(NQ):
                    s = pl.ds(q * L, L)
                    buf[e, s] = buf[e, s] * w

    def run_layer(src_tbl, acc):
        def issue_gather(j, b):
            pltpu.async_copy(src_tbl.at[src_v.at[j]], gbufs[b], gsems[b])

        def wait_gather(b):
            pltpu.make_async_copy(src_tbl.at[src_v.at[0]], gbufs[b],
                                  gsems[b]).wait()

        def issue_scatter(j, b):
            pltpu.async_copy(gbufs[b], acc.at[dst_v.at[j]], ssems[b],
                             add=True)

        def wait_scatter(b):
            pltpu.make_async_copy(gbufs[b], acc.at[dst_v.at[0]],
                                  ssems[b]).wait()

        def superchunk(s, carry):
            ms = pl.ds(s * SCH, SCH)
            pltpu.sync_copy(srcm.at[sid, ms], src_v)
            pltpu.sync_copy(dstm.at[sid, ms], dst_v)
            pltpu.sync_copy(wm.at[sid, ms], w_v)
            for p in range(LA):
                issue_gather(p, p)

            def group(g, carry2):
                for b in range(NBUF):
                    j = g * NBUF + b
                    wait_gather(b)
                    nb = (b + LA) % NBUF
                    if b < NBUF - LA:
                        @pl.when(g > 0)
                        def _():
                            wait_scatter(nb)

                        issue_gather(j + LA, nb)
                    else:
                        @pl.when(g < GRP - 1)
                        def _():
                            wait_scatter(nb)
                            issue_gather(j + LA, nb)

                    scale_chunk(j, b)
                    issue_scatter(j, b)
                return carry2

            lax.fori_loop(0, GRP, group, 0)
            for b in range(NBUF):
                wait_scatter(b)
            return carry

        lax.fori_loop(0, NSC, superchunk, 0)

    run_layer(acc2, acc1)
    plsc.subcore_barrier()

    def zacc2(rs):
        pltpu.sync_copy(cb0, acc2.at[rs])

    for_blocks(zacc2)
    plsc.subcore_barrier()

    run_layer(acc1, acc2)
    plsc.subcore_barrier()

    def combine(rs):
        pltpu.sync_copy(tbl.at[cid].at[rs], cb0)
        pltpu.sync_copy(acc1.at[rs], cb1)

        def arow(r, carry):
            for q in range(NQ):
                s = pl.ds(q * L, L)
                cb0[r, s] = cb0[r, s] + cb1[r, s]
            return carry

        lax.fori_loop(0, CR, arow, 0)
        pltpu.sync_copy(acc2.at[rs], cb1)
        lax.fori_loop(0, CR, arow, 0)
        pltpu.sync_copy(cb0, out.at[cid].at[rs])

    for_blocks(combine)


@jax.jit
def _run(tbl, srcm, dstm, wm):
    return pl.kernel(
        _body,
        out_type=jax.ShapeDtypeStruct((NC, N, DH), jnp.float32),
        mesh=plsc.VectorSubcoreMesh(core_axis_name="c", subcore_axis_name="s"),
        compiler_params=pltpu.CompilerParams(use_tc_tiling_on_sc=False),
        scratch_types=[
            pltpu.VMEM((SCH, K), jnp.int32),
            pltpu.VMEM((SCH, K), jnp.int32),
            pltpu.VMEM((SCH, K), jnp.float32),
            pltpu.VMEM((K, DH), jnp.float32),
            pltpu.VMEM((K, DH), jnp.float32),
            pltpu.VMEM((K, DH), jnp.float32),
            pltpu.VMEM((K, DH), jnp.float32),
            pltpu.VMEM((K, DH), jnp.float32),
            pltpu.VMEM((K, DH), jnp.float32),
            pltpu.VMEM((CR, DH), jnp.float32),
            pltpu.VMEM((CR, DH), jnp.float32),
            pltpu.VMEM_SHARED((N, DH), jnp.float32),
            pltpu.VMEM_SHARED((N, DH), jnp.float32),
            pltpu.SemaphoreType.DMA,
            pltpu.SemaphoreType.DMA,
            pltpu.SemaphoreType.DMA,
            pltpu.SemaphoreType.DMA,
            pltpu.SemaphoreType.DMA,
            pltpu.SemaphoreType.DMA,
            pltpu.SemaphoreType.DMA,
            pltpu.SemaphoreType.DMA,
            pltpu.SemaphoreType.DMA,
            pltpu.SemaphoreType.DMA,
            pltpu.SemaphoreType.DMA,
            pltpu.SemaphoreType.DMA,
        ],
    )(tbl, srcm, dstm, wm)


def kernel(edge_index, edge_weight, uEmbeds, iEmbeds):
    embeds = jnp.concatenate([uEmbeds, iEmbeds], axis=0)
    tbl = jnp.stack([embeds[:, :DH], embeds[:, DH:]], axis=0)
    pad = E_PAD - E
    src = jnp.concatenate(
        [edge_index[1], jnp.zeros((pad,), jnp.int32)]).reshape(NS, CHT, K)
    dst = jnp.concatenate(
        [edge_index[0], jnp.zeros((pad,), jnp.int32)]).reshape(NS, CHT, K)
    w = jnp.concatenate(
        [edge_weight, jnp.zeros((pad,), jnp.float32)]).reshape(NS, CHT, K)
    out = _run(tbl, src, dst, w)
    full = jnp.concatenate([out[0], out[1]], axis=1)
    return full[:N_USER], full[N_USER:]


# speedup vs baseline: 1.0013x; 1.0013x over previous
import jax
import jax.numpy as jnp
from jax import lax
from jax.experimental import pallas as pl
from jax.experimental.pallas import tpu as pltpu
from jax.experimental.pallas import tpu_sc as plsc

N_USER = 5000
N_ITEM = 5000
N = N_USER + N_ITEM
D = 128
DH = 64
E = 320000
NC = 2
NS = 16
L = 16

K = 80
NBUF = 4
CHT = 252
SCH = 84
NSC = CHT // SCH
GRP = SCH // NBUF
E_PAD = NS * CHT * K
CR = 80
NB = N // CR
NQ = DH // L


def _body(tbl, srcm, dstm, wm, out, mir,
          src_v, dst_v, w_v, gb0, gb1, gb2, gb3, cb0, cb1, acc1, acc2,
          gs0, gs1, gs2, gs3, ss0, ss1, ss2, ss3):
    cid = lax.axis_index("c")
    sid = lax.axis_index("s")
    gbufs = (gb0, gb1, gb2, gb3)
    gsems = (gs0, gs1, gs2, gs3)
    ssems = (ss0, ss1, ss2, ss3)

    zero16 = jnp.zeros((L,), jnp.float32)

    def zrow(r, carry):
        for q in range(NQ):
            cb0[r, pl.ds(q * L, L)] = zero16
        return carry

    lax.fori_loop(0, CR, zrow, 0)

    def for_blocks(fn):
        for t in range((NB + NS - 1) // NS):
            bb = t * NS + sid

            @pl.when(bb < NB)
            def _():
                fn(pl.ds(bb * CR, CR))

    def stage0(rs):
        pltpu.sync_copy(tbl.at[cid].at[rs], acc2.at[rs])
        pltpu.sync_copy(cb0, acc1.at[rs])

    for_blocks(stage0)
    plsc.subcore_barrier()

    def scale_chunk(j, b):
        buf = gbufs[b]
        for g in range(K // L):
            wvec = w_v[j, pl.ds(g * L, L)]
            for l in range(L):
                w = wvec[l]
                e = g * L + l
                for q in range(NQ):
                    s = pl.ds(q * L, L)
                    buf[e, s] = buf[e, s] * w

    def run_layer(src_tbl, src_hbm, acc):
        def issue_gather(j, b):
            if b % 2 == 0:
                pltpu.async_copy(src_hbm.at[src_v.at[j]], gbufs[b], gsems[b])
            else:
                pltpu.async_copy(src_tbl.at[src_v.at[j]], gbufs[b], gsems[b])

        def wait_gather(b):
            pltpu.make_async_copy(src_tbl.at[src_v.at[0]], gbufs[b],
                                  gsems[b]).wait()

        def issue_scatter(j, b):
            pltpu.async_copy(gbufs[b], acc.at[dst_v.at[j]], ssems[b],
                             add=True)

        def wait_scatter(b):
            pltpu.make_async_copy(gbufs[b], acc.at[dst_v.at[0]],
                                  ssems[b]).wait()

        def superchunk(s, carry):
            ms = pl.ds(s * SCH, SCH)
            pltpu.sync_copy(srcm.at[sid, ms], src_v)
            pltpu.sync_copy(dstm.at[sid, ms], dst_v)
            pltpu.sync_copy(wm.at[sid, ms], w_v)
            issue_gather(0, 0)
            issue_gather(1, 1)

            def group(g, carry2):
                for b in range(NBUF):
                    j = g * NBUF + b
                    wait_gather(b)
                    nb = (b + 2) % NBUF
                    if b < 2:
                        @pl.when(g > 0)
                        def _():
                            wait_scatter(nb)

                        issue_gather(j + 2, nb)
                    else:
                        @pl.when(g < GRP - 1)
                        def _():
                            wait_scatter(nb)
                            issue_gather(j + 2, nb)

                    scale_chunk(j, b)
                    issue_scatter(j, b)
                return carry2

            lax.fori_loop(0, GRP, group, 0)
            for b in range(NBUF):
                wait_scatter(b)
            return carry

        lax.fori_loop(0, NSC, superchunk, 0)

    run_layer(acc2, tbl.at[cid], acc1)
    plsc.subcore_barrier()

    def zacc2(rs):
        pltpu.sync_copy(cb0, acc2.at[rs])
        pltpu.sync_copy(acc1.at[rs], mir.at[cid].at[rs])

    for_blocks(zacc2)
    plsc.subcore_barrier()

    run_layer(acc1, mir.at[cid], acc2)
    plsc.subcore_barrier()

    def combine(rs):
        pltpu.sync_copy(tbl.at[cid].at[rs], cb0)
        pltpu.sync_copy(acc1.at[rs], cb1)

        def arow(r, carry):
            for q in range(NQ):
                s = pl.ds(q * L, L)
                cb0[r, s] = cb0[r, s] + cb1[r, s]
            return carry

        lax.fori_loop(0, CR, arow, 0)
        pltpu.sync_copy(acc2.at[rs], cb1)
        lax.fori_loop(0, CR, arow, 0)
        pltpu.sync_copy(cb0, out.at[cid].at[rs])

    for_blocks(combine)


@jax.jit
def _run(tbl, srcm, dstm, wm):
    return pl.kernel(
        _body,
        out_type=(jax.ShapeDtypeStruct((NC, N, DH), jnp.float32),
                  jax.ShapeDtypeStruct((NC, N, DH), jnp.float32)),
        mesh=plsc.VectorSubcoreMesh(core_axis_name="c", subcore_axis_name="s"),
        compiler_params=pltpu.CompilerParams(use_tc_tiling_on_sc=False),
        scratch_types=[
            pltpu.VMEM((SCH, K), jnp.int32),
            pltpu.VMEM((SCH, K), jnp.int32),
            pltpu.VMEM((SCH, K), jnp.float32),
            pltpu.VMEM((K, DH), jnp.float32),
            pltpu.VMEM((K, DH), jnp.float32),
            pltpu.VMEM((K, DH), jnp.float32),
            pltpu.VMEM((K, DH), jnp.float32),
            pltpu.VMEM((CR, DH), jnp.float32),
            pltpu.VMEM((CR, DH), jnp.float32),
            pltpu.VMEM_SHARED((N, DH), jnp.float32),
            pltpu.VMEM_SHARED((N, DH), jnp.float32),
            pltpu.SemaphoreType.DMA,
            pltpu.SemaphoreType.DMA,
            pltpu.SemaphoreType.DMA,
            pltpu.SemaphoreType.DMA,
            pltpu.SemaphoreType.DMA,
            pltpu.SemaphoreType.DMA,
            pltpu.SemaphoreType.DMA,
            pltpu.SemaphoreType.DMA,
        ],
    )(tbl, srcm, dstm, wm)


def kernel(edge_index, edge_weight, uEmbeds, iEmbeds):
    embeds = jnp.concatenate([uEmbeds, iEmbeds], axis=0)
    tbl = jnp.stack([embeds[:, :DH], embeds[:, DH:]], axis=0)
    pad = E_PAD - E
    src = jnp.concatenate(
        [edge_index[1], jnp.zeros((pad,), jnp.int32)]).reshape(NS, CHT, K)
    dst = jnp.concatenate(
        [edge_index[0], jnp.zeros((pad,), jnp.int32)]).reshape(NS, CHT, K)
    w = jnp.concatenate(
        [edge_weight, jnp.zeros((pad,), jnp.float32)]).reshape(NS, CHT, K)
    out, _ = _run(tbl, src, dst, w)
    full = jnp.concatenate([out[0], out[1]], axis=1)
    return full[:N_USER], full[N_USER:]


# speedup vs baseline: 1.2381x; 1.2365x over previous
import jax
import jax.numpy as jnp
from jax import lax
from jax.experimental import pallas as pl
from jax.experimental.pallas import tpu as pltpu
from jax.experimental.pallas import tpu_sc as plsc

N_USER = 5000
N_ITEM = 5000
N = N_USER + N_ITEM
D = 128
DH = 64
E = 320000
NC = 2
NS = 16
L = 16
LW = 32

K = 80
NBUF = 4
LA = 2
CHT = 252
SCH = 36
NSC = CHT // SCH
GRP = SCH // NBUF
E_PAD = NS * CHT * K
CR = 80
NB = N // CR
NQ = DH // LW
ILV = plsc.PackFormat.INTERLEAVED


def _body(tblh, tblf, srcm, dstm, wm, out,
          src_v, dst_v, w_v, gb0, gb1, gb2, gb3, sb0, sb1, sb2, sb3,
          cb0, cb1h, ixb, tabh, acc1, acc2,
          gs0, gs1, gs2, gs3, ss0, ss1, ss2, ss3):
    cid = lax.axis_index("c")
    sid = lax.axis_index("s")
    gbufs = (gb0, gb1, gb2, gb3)
    sbufs = (sb0, sb1, sb2, sb3)
    gsems = (gs0, gs1, gs2, gs3)
    ssems = (ss0, ss1, ss2, ss3)

    zero16 = jnp.zeros((L,), jnp.float32)
    zero32 = jnp.zeros((LW,), jnp.bfloat16)

    def zrow(r, carry):
        for q in range(NQ):
            cb0[r, pl.ds(2 * q * L, L)] = zero16
            cb0[r, pl.ds((2 * q + 1) * L, L)] = zero16
            cb1h[r, pl.ds(q * LW, LW)] = zero32
        return carry

    lax.fori_loop(0, CR, zrow, 0)

    def for_blocks(fn):
        for t in range((NB + NS - 1) // NS):
            bb = t * NS + sid

            @pl.when(bb < NB)
            def _():
                fn(pl.ds(bb * CR, CR), bb * CR)

    def stage0(rs, base):
        pltpu.sync_copy(tblh.at[cid].at[rs], tabh.at[rs])
        pltpu.sync_copy(cb1h, acc1.at[rs])
        pltpu.sync_copy(cb0, acc2.at[rs])

    for_blocks(stage0)
    plsc.subcore_barrier()

    def scale_bf16(j, b):
        buf = gbufs[b]
        for g in range(K // L):
            wvec = w_v[j, pl.ds(g * L, L)]
            for l in range(L):
                w = wvec[l]
                e = g * L + l
                for q in range(NQ):
                    s = pl.ds(q * LW, LW)
                    va, vb = plsc.unpack(buf[e, s], format=ILV)
                    buf[e, s] = plsc.pack(va * w, vb * w, format=ILV)

    def scale_f32(j, b):
        gbuf = gbufs[b]
        sbuf = sbufs[b]
        for g in range(K // L):
            wvec = w_v[j, pl.ds(g * L, L)]
            for l in range(L):
                w = wvec[l]
                e = g * L + l
                for q in range(NQ):
                    va, vb = plsc.unpack(gbuf[e, pl.ds(q * LW, LW)],
                                         format=ILV)
                    sbuf[e, pl.ds(2 * q * L, L)] = va * w
                    sbuf[e, pl.ds((2 * q + 1) * L, L)] = vb * w

    def run_layer(src_tbl, acc, scale, scat_bufs):
        def issue_gather(j, b):
            pltpu.async_copy(src_tbl.at[src_v.at[j]], gbufs[b], gsems[b])

        def wait_gather(b):
            pltpu.make_async_copy(src_tbl.at[src_v.at[0]], gbufs[b],
                                  gsems[b]).wait()

        def issue_scatter(j, b):
            pltpu.async_copy(scat_bufs[b], acc.at[dst_v.at[j]], ssems[b],
                             add=True)

        def wait_scatter(b):
            pltpu.make_async_copy(scat_bufs[b], acc.at[dst_v.at[0]],
                                  ssems[b]).wait()

        def superchunk(s, carry):
            ms = pl.ds(s * SCH, SCH)
            pltpu.sync_copy(srcm.at[sid, ms], src_v)
            pltpu.sync_copy(dstm.at[sid, ms], dst_v)
            pltpu.sync_copy(wm.at[sid, ms], w_v)
            for p in range(LA):
                issue_gather(p, p)

            def group(g, carry2):
                for b in range(NBUF):
                    j = g * NBUF + b
                    wait_gather(b)
                    nb = (b + LA) % NBUF
                    if b < NBUF - LA:
                        @pl.when(g > 0)
                        def _():
                            wait_scatter(nb)

                        issue_gather(j + LA, nb)
                    else:
                        @pl.when(g < GRP - 1)
                        def _():
                            wait_scatter(nb)
                            issue_gather(j + LA, nb)

                    scale(j, b)
                    issue_scatter(j, b)
                return carry2

            lax.fori_loop(0, GRP, group, 0)
            for b in range(NBUF):
                wait_scatter(b)
            return carry

        lax.fori_loop(0, NSC, superchunk, 0)

    run_layer(tabh, acc1, scale_bf16, gbufs)
    plsc.subcore_barrier()

    run_layer(acc1, acc2, scale_f32, sbufs)
    plsc.subcore_barrier()

    def combine(rs, base):
        pltpu.sync_copy(tblf.at[cid].at[rs], cb0)
        pltpu.sync_copy(acc1.at[rs], cb1h)

        def arow1(r, carry):
            for q in range(NQ):
                va, vb = plsc.unpack(cb1h[r, pl.ds(q * LW, LW)], format=ILV)
                sa = pl.ds(2 * q * L, L)
                sb = pl.ds((2 * q + 1) * L, L)
                cb0[r, sa] = cb0[r, sa] + va
                cb0[r, sb] = cb0[r, sb] + vb
            return carry

        lax.fori_loop(0, CR, arow1, 0)
        pltpu.sync_copy(acc2.at[rs], sb0)

        def arow2(r, carry):
            for q in range(NQ):
                for h in range(2):
                    s = pl.ds((2 * q + h) * L, L)
                    cb0[r, s] = cb0[r, s] + sb0[r, s]
            return carry

        lax.fori_loop(0, CR, arow2, 0)
        pltpu.sync_copy(cb0, out.at[cid].at[rs])

    for_blocks(combine)


@jax.jit
def _run(tblh, tblf, srcm, dstm, wm):
    return pl.kernel(
        _body,
        out_type=jax.ShapeDtypeStruct((NC, N, DH), jnp.float32),
        mesh=plsc.VectorSubcoreMesh(core_axis_name="c", subcore_axis_name="s"),
        compiler_params=pltpu.CompilerParams(use_tc_tiling_on_sc=False,
                                             needs_layout_passes=False),
        scratch_types=[
            pltpu.VMEM((SCH, K), jnp.int32),
            pltpu.VMEM((SCH, K), jnp.int32),
            pltpu.VMEM((SCH, K), jnp.float32),
            pltpu.VMEM((K, DH), jnp.bfloat16),
            pltpu.VMEM((K, DH), jnp.bfloat16),
            pltpu.VMEM((K, DH), jnp.bfloat16),
            pltpu.VMEM((K, DH), jnp.bfloat16),
            pltpu.VMEM((K, DH), jnp.float32),
            pltpu.VMEM((K, DH), jnp.float32),
            pltpu.VMEM((K, DH), jnp.float32),
            pltpu.VMEM((K, DH), jnp.float32),
            pltpu.VMEM((CR, DH), jnp.float32),
            pltpu.VMEM((CR, DH), jnp.bfloat16),
            pltpu.VMEM((CR,), jnp.int32),
            pltpu.VMEM_SHARED((N, DH), jnp.bfloat16),
            pltpu.VMEM_SHARED((N, DH), jnp.bfloat16),
            pltpu.VMEM_SHARED((N, DH), jnp.float32),
            pltpu.SemaphoreType.DMA,
            pltpu.SemaphoreType.DMA,
            pltpu.SemaphoreType.DMA,
            pltpu.SemaphoreType.DMA,
            pltpu.SemaphoreType.DMA,
            pltpu.SemaphoreType.DMA,
            pltpu.SemaphoreType.DMA,
            pltpu.SemaphoreType.DMA,
        ],
    )(tblh, tblf, srcm, dstm, wm)


def _interleave_cols(x):
    n, d = x.shape
    return (x.reshape(n, d // LW, 2, L)
             .transpose(0, 1, 3, 2)
             .reshape(n, d))


def kernel(edge_index, edge_weight, uEmbeds, iEmbeds):
    embeds = jnp.concatenate([uEmbeds, iEmbeds], axis=0)
    tblf = jnp.stack([embeds[:, :DH], embeds[:, DH:]], axis=0)
    eh = _interleave_cols(embeds.astype(jnp.bfloat16))
    tblh = jnp.stack([eh[:, :DH], eh[:, DH:]], axis=0)
    pad = E_PAD - E
    src = jnp.concatenate(
        [edge_index[1], jnp.zeros((pad,), jnp.int32)]).reshape(NS, CHT, K)
    dst = jnp.concatenate(
        [edge_index[0], jnp.zeros((pad,), jnp.int32)]).reshape(NS, CHT, K)
    w = jnp.concatenate(
        [edge_weight, jnp.zeros((pad,), jnp.float32)]).reshape(NS, CHT, K)
    out = _run(tblh, tblf, src, dst, w)
    full = jnp.concatenate([out[0], out[1]], axis=1)
    return full[:N_USER], full[N_USER:]
